# baseline (speedup 1.0000x reference)
import jax
import jax.numpy as jnp
from jax import lax
from jax.experimental import pallas as pl
from jax.experimental.pallas import tpu as pltpu
from jax.experimental.pallas import tpu_sc as plsc

N = 10000
E = 320000
D = 128

NC = 2
NS = 16
NW = NC * NS
EPW = E // NW
CHUNK = 80
NCHUNK = EPW // CHUNK
NP = 10240
ROWS_PER_TILE = NP // NS

NRB = 4
NIB = 8


def _matmul_kernel(x_ref, w_ref, b_ref, o_ref):
    o_ref[...] = (
        jnp.dot(x_ref[...], w_ref[...], preferred_element_type=jnp.float32)
        + b_ref[...]
    )


def _add_kernel(a_ref, b_ref, o_ref):
    o_ref[...] = a_ref[...] + b_ref[...]


def _spmm_body(h_hbm, src_hbm, dst_hbm, val_hbm, out_hbm,
               srcb, dstb, valb, rows, zbuf, acc_shared, srs, dss, vls, gs, ss):
    cid = lax.axis_index("c")
    sid = lax.axis_index("s")
    wid = sid * NC + cid

    ebase = wid * EPW
    LAST = NCHUNK - 1

    def start_sv(c, m):
        off = ebase + c * CHUNK
        pltpu.async_copy(src_hbm.at[pl.ds(off, CHUNK)], srcb[m], srs[m])
        pltpu.async_copy(val_hbm.at[pl.ds(off, CHUNK)], valb[m], vls[m])

    def wait_sv(c, m):
        off = ebase + c * CHUNK
        pltpu.make_async_copy(src_hbm.at[pl.ds(off, CHUNK)], srcb[m], srs[m]).wait()
        pltpu.make_async_copy(val_hbm.at[pl.ds(off, CHUNK)], valb[m], vls[m]).wait()

    def start_dst(c, m):
        off = ebase + c * CHUNK
        pltpu.async_copy(dst_hbm.at[pl.ds(off, CHUNK)], dstb[m], dss[m])

    def wait_dst(c, m):
        off = ebase + c * CHUNK
        pltpu.make_async_copy(dst_hbm.at[pl.ds(off, CHUNK)], dstb[m], dss[m]).wait()

    def start_gather(msrc, mdst, sem):
        pltpu.async_copy(h_hbm.at[srcb[msrc]], rows[mdst], sem)

    def wait_gather(m):
        pltpu.make_async_copy(h_hbm.at[srcb[m]], rows[m], gs[m]).wait()

    def start_scatter(m):
        pltpu.async_copy(rows[m], acc_shared.at[dstb[m]], ss[m], add=True)

    def wait_scatter(m):
        pltpu.make_async_copy(h_hbm.at[pl.ds(0, CHUNK)], rows[m], ss[m]).wait()

    def scale(m):
        buf = rows[m]
        vref = valb[m]

        def group_body(g, carry):
            gbase = g * 16
            vv = vref[pl.ds(gbase, 16)]
            for e in range(16):
                v = vv[e]
                for j in range(D // 16):
                    sl = pl.ds(j * 16, 16)
                    buf[gbase + e, sl] = buf[gbase + e, sl] * v
            return carry

        lax.fori_loop(0, CHUNK // 16, group_body, 0)

    def step(c, k):
        kn = (k + 2) % 4
        cg = jnp.minimum(c + 2, LAST)
        ci = jnp.minimum(c + 4, LAST)
        wait_gather(k)
        wait_dst(c, k)
        wait_scatter(kn)
        wait_sv(cg, kn)
        start_gather(kn, kn, gs[kn])
        start_dst(cg, kn)
        scale(k)
        start_scatter(k)
        start_sv(ci, k)

    for c in range(4):
        start_sv(c, c)
    start_dst(0, 0)
    start_dst(1, 1)
    wait_sv(0, 0)
    start_gather(0, 0, gs[0])
    start_gather(0, 2, ss[2])
    wait_sv(1, 1)
    start_gather(1, 1, gs[1])
    start_gather(1, 3, ss[3])

    zv = jnp.zeros((16,), jnp.float32)

    def zrow(r, carry):
        for j in range(D // 16):
            zbuf[r, pl.ds(j * 16, 16)] = zv
        return carry

    lax.fori_loop(0, 32, zrow, 0)
    for t in range(ROWS_PER_TILE // 32):
        pltpu.sync_copy(
            zbuf,
            acc_shared.at[pl.ds(sid * ROWS_PER_TILE + t * 32, 32)],
        )
    plsc.subcore_barrier()

    def quad_body(p, carry):
        base = 4 * p
        for k in range(4):
            step(base + k, k)
        return carry

    lax.fori_loop(0, NCHUNK // 4, quad_body, 0)
    step(LAST, 0)

    wait_gather(1)
    wait_gather(2)
    wait_sv(LAST, 3)
    wait_sv(LAST, 0)
    wait_dst(LAST, 1)
    wait_dst(LAST, 2)
    wait_scatter(3)
    wait_scatter(0)

    plsc.subcore_barrier()

    pltpu.sync_copy(
        acc_shared.at[pl.ds(sid * ROWS_PER_TILE, ROWS_PER_TILE)],
        out_hbm.at[cid, pl.ds(sid * ROWS_PER_TILE, ROWS_PER_TILE)],
    )


@jax.jit
def _spmm(h, src1, dst1, val1):
    mesh = plsc.VectorSubcoreMesh(core_axis_name="c", subcore_axis_name="s")
    f = pl.kernel(
        _spmm_body,
        out_type=jax.ShapeDtypeStruct((NC, NP, D), jnp.float32),
        mesh=mesh,
        scratch_types=[
            [pltpu.VMEM((CHUNK,), jnp.int32) for _ in range(4)],
            [pltpu.VMEM((CHUNK,), jnp.int32) for _ in range(4)],
            [pltpu.VMEM((CHUNK,), jnp.float32) for _ in range(4)],
            [pltpu.VMEM((CHUNK, D), jnp.float32) for _ in range(4)],
            pltpu.VMEM((32, D), jnp.float32),
            pltpu.VMEM_SHARED((NP, D), jnp.float32),
            [pltpu.SemaphoreType.DMA for _ in range(4)],
            [pltpu.SemaphoreType.DMA for _ in range(4)],
            [pltpu.SemaphoreType.DMA for _ in range(4)],
            [pltpu.SemaphoreType.DMA for _ in range(4)],
            [pltpu.SemaphoreType.DMA for _ in range(4)],
        ],
    )
    return f(h, src1, dst1, val1)


def kernel(x, adj_indices, adj_values, W, b):
    h = pl.pallas_call(
        _matmul_kernel,
        grid=(10,),
        in_specs=[
            pl.BlockSpec((N // 10, D), lambda i: (i, 0)),
            pl.BlockSpec((D, D), lambda i: (0, 0)),
            pl.BlockSpec((1, D), lambda i: (0, 0)),
        ],
        out_specs=pl.BlockSpec((N // 10, D), lambda i: (i, 0)),
        out_shape=jax.ShapeDtypeStruct((N, D), jnp.float32),
    )(x, W, b.reshape(1, D))

    dst1 = adj_indices[0]
    src1 = adj_indices[1]
    partials = _spmm(h, src1, dst1, adj_values)

    out = pl.pallas_call(
        _add_kernel,
        grid=(10,),
        in_specs=[
            pl.BlockSpec((N // 10, D), lambda i: (i, 0)),
            pl.BlockSpec((N // 10, D), lambda i: (i, 0)),
        ],
        out_specs=pl.BlockSpec((N // 10, D), lambda i: (i, 0)),
        out_shape=jax.ShapeDtypeStruct((N, D), jnp.float32),
    )(partials[0], partials[1])
    return out
